# Initial kernel scaffold; baseline (speedup 1.0000x reference)
import jax
import jax.numpy as jnp
from jax.experimental import pallas as pl


def kernel(x, edge_index, batch, W0, b0, Ws, bs, gammas, betas, Wf, bf):
    raise NotImplementedError("write your pallas kernel here")


# speedup vs baseline: 4.9086x; 4.9086x over previous
import functools

import jax
import jax.numpy as jnp
from jax import lax
from jax.experimental import pallas as pl
from jax.experimental.pallas import tpu as pltpu
from jax.experimental.pallas import tpu_sc as plsc

_N = 10000
_E = 320000
_D = 128
_G = 64
_NL = 8

_NP = 10240
_CH = 128
_NCH = 80
_IST = 40
_TILES = 32
_EP = _TILES * _NCH * _CH
_STRIPE = _NP // 16

_f32 = jnp.float32


def _agg_body(hp_hbm, src_hbm, dst_hbm, out_hbm,
              src_v, dst_v, rows_a, rows_b, acc_sh, sem_a, sem_b):
    cid = lax.axis_index("c")
    sid = lax.axis_index("s")
    wid = cid * 16 + sid

    zero16 = jnp.zeros((16,), _f32)

    def zrow(r, carry):
        for l in range(_D // 16):
            rows_a[r, pl.ds(l * 16, 16)] = zero16
        return carry

    lax.fori_loop(0, _CH, zrow, 0)
    base = sid * _STRIPE
    for k in range(_STRIPE // _CH):
        pltpu.sync_copy(rows_a, acc_sh.at[pl.ds(base + k * _CH, _CH)])

    plsc.subcore_barrier()

    for t in range(_NCH // _IST):
        pltpu.sync_copy(src_hbm.at[wid, pl.ds(t * _IST, _IST)], src_v)
        pltpu.sync_copy(dst_hbm.at[wid, pl.ds(t * _IST, _IST)], dst_v)

        def body(jj, carry):
            ca = pltpu.async_copy(hp_hbm.at[src_v.at[2 * jj]], rows_a, sem_a)
            cb = pltpu.async_copy(hp_hbm.at[src_v.at[2 * jj + 1]], rows_b,
                                  sem_b)
            ca.wait()
            pltpu.sync_copy(rows_a, acc_sh.at[dst_v.at[2 * jj]], add=True)
            cb.wait()
            pltpu.sync_copy(rows_b, acc_sh.at[dst_v.at[2 * jj + 1]],
                            add=True)
            return carry

        lax.fori_loop(0, _IST // 2, body, 0)

    plsc.subcore_barrier()
    pltpu.sync_copy(acc_sh.at[pl.ds(base, _STRIPE)],
                    out_hbm.at[cid, pl.ds(base, _STRIPE)])


_agg_call = functools.partial(
    pl.kernel,
    out_type=jax.ShapeDtypeStruct((2, _NP, _D), _f32),
    mesh=plsc.VectorSubcoreMesh(core_axis_name="c", subcore_axis_name="s"),
    scratch_types=[
        pltpu.VMEM((_IST, _CH), jnp.int32),
        pltpu.VMEM((_IST, _CH), jnp.int32),
        pltpu.VMEM((_CH, _D), _f32),
        pltpu.VMEM((_CH, _D), _f32),
        pltpu.VMEM_SHARED((_NP, _D), _f32),
        pltpu.SemaphoreType.DMA,
        pltpu.SemaphoreType.DMA,
    ],
)(_agg_body)


_RB = 256


def _init_body(dacc_ref, x_ref, w_ref, b_ref, hp_ref, dinv_ref):
    i = pl.program_id(0)
    deg = dacc_ref[0, :, 0] + dacc_ref[1, :, 0] + 1.0
    di = lax.rsqrt(jnp.maximum(deg, 1.0))
    rows = i * _RB + lax.broadcasted_iota(jnp.int32, (_RB,), 0)
    di = jnp.where(rows < _N, di, 0.0)
    h0 = jnp.dot(x_ref[...], w_ref[...],
                 preferred_element_type=_f32) + b_ref[...]
    dinv_ref[...] = jnp.broadcast_to(di[:, None], (_RB, _D))
    hp_ref[...] = h0 * di[:, None]


def _init_call(dacc, x_p, w0, b0):
    return pl.pallas_call(
        _init_body,
        grid=(_NP // _RB,),
        in_specs=[
            pl.BlockSpec((2, _RB, _D), lambda i: (0, i, 0)),
            pl.BlockSpec((_RB, _D), lambda i: (i, 0)),
            pl.BlockSpec((_D, _D), lambda i: (0, 0)),
            pl.BlockSpec((1, _D), lambda i: (0, 0)),
        ],
        out_specs=[
            pl.BlockSpec((_RB, _D), lambda i: (i, 0)),
            pl.BlockSpec((_RB, _D), lambda i: (i, 0)),
        ],
        out_shape=[
            jax.ShapeDtypeStruct((_NP, _D), _f32),
            jax.ShapeDtypeStruct((_NP, _D), _f32),
        ],
    )(dacc, x_p, w0, b0)


def _layer_body(acc_ref, hp_ref, dinv_ref, w_ref, b_ref, g_ref, be_ref,
                hp_out_ref, h_out_ref):
    di = dinv_ref[...]
    u = di * (acc_ref[0] + acc_ref[1] + hp_ref[...])
    z = jnp.dot(u, w_ref[...], preferred_element_type=_f32) + b_ref[...]
    mu = jnp.mean(z, axis=-1, keepdims=True)
    zc = z - mu
    var = jnp.mean(zc * zc, axis=-1, keepdims=True)
    h = zc * lax.rsqrt(var + 1e-5) * g_ref[...] + be_ref[...]
    h = jnp.maximum(h, 0.0)
    h_out_ref[...] = h
    hp_out_ref[...] = h * di


def _layer_call(acc, hp, dinv, w, b, g, be):
    return pl.pallas_call(
        _layer_body,
        grid=(_NP // _RB,),
        in_specs=[
            pl.BlockSpec((2, _RB, _D), lambda i: (0, i, 0)),
            pl.BlockSpec((_RB, _D), lambda i: (i, 0)),
            pl.BlockSpec((_RB, _D), lambda i: (i, 0)),
            pl.BlockSpec((_D, _D), lambda i: (0, 0)),
            pl.BlockSpec((1, _D), lambda i: (0, 0)),
            pl.BlockSpec((1, _D), lambda i: (0, 0)),
            pl.BlockSpec((1, _D), lambda i: (0, 0)),
        ],
        out_specs=[
            pl.BlockSpec((_RB, _D), lambda i: (i, 0)),
            pl.BlockSpec((_RB, _D), lambda i: (i, 0)),
        ],
        out_shape=[
            jax.ShapeDtypeStruct((_NP, _D), _f32),
            jax.ShapeDtypeStruct((_NP, _D), _f32),
        ],
    )(acc, hp, dinv, w, b, g, be)


_RP = 400


def _pool_body(batch_ref, h_ref, wf_ref, bf_ref, out_ref, sums, counts):
    i = pl.program_id(0)

    @pl.when(i == 0)
    def _():
        sums[...] = jnp.zeros_like(sums)
        counts[...] = jnp.zeros_like(counts)

    b = batch_ref[...][:, 0]
    oh = (lax.broadcasted_iota(jnp.int32, (_G, _RP), 0)
          == b[None, :]).astype(_f32)
    sums[...] += jnp.dot(oh, h_ref[...], preferred_element_type=_f32)
    counts[...] += jnp.sum(oh, axis=1, keepdims=True)

    @pl.when(i == _N // _RP - 1)
    def _():
        pooled = sums[...] / jnp.maximum(counts[...], 1.0)
        out_ref[...] = jnp.dot(pooled, wf_ref[...],
                               preferred_element_type=_f32) + bf_ref[...]


def _pool_call(batch2, h, wf, bf):
    return pl.pallas_call(
        _pool_body,
        grid=(_N // _RP,),
        in_specs=[
            pl.BlockSpec((_RP, 1), lambda i: (i, 0)),
            pl.BlockSpec((_RP, _D), lambda i: (i, 0)),
            pl.BlockSpec((_D, _D), lambda i: (0, 0)),
            pl.BlockSpec((1, _D), lambda i: (0, 0)),
        ],
        out_specs=pl.BlockSpec((_G, _D), lambda i: (0, 0)),
        out_shape=jax.ShapeDtypeStruct((_G, _D), _f32),
        scratch_shapes=[
            pltpu.VMEM((_G, _D), _f32),
            pltpu.VMEM((_G, _D), _f32),
        ],
    )(batch2, h, wf, bf)


def kernel(x, edge_index, batch, W0, b0, Ws, bs, gammas, betas, Wf, bf):
    src = edge_index[0]
    dst = edge_index[1]
    pad = jnp.full((_EP - _E,), _N, jnp.int32)
    src_p = jnp.concatenate([src, pad]).reshape(_TILES, _NCH, _CH)
    dst_p = jnp.concatenate([dst, pad]).reshape(_TILES, _NCH, _CH)
    x_p = jnp.concatenate([x, jnp.zeros((_NP - _N, _D), _f32)], axis=0)
    ones_p = jnp.concatenate([jnp.ones((_N, _D), _f32),
                              jnp.zeros((_NP - _N, _D), _f32)], axis=0)

    dacc = _agg_call(ones_p, src_p, dst_p)
    hp, dinv = _init_call(dacc, x_p, W0, b0.reshape(1, _D))
    h = None
    for i in range(_NL):
        acc = _agg_call(hp, src_p, dst_p)
        hp, h = _layer_call(acc, hp, dinv, Ws[i], bs[i].reshape(1, _D),
                            gammas[i].reshape(1, _D),
                            betas[i].reshape(1, _D))
    return _pool_call(batch.reshape(_N, 1), h, Wf, bf.reshape(1, _D))
